# Initial kernel scaffold; baseline (speedup 1.0000x reference)
import jax
import jax.numpy as jnp
from jax.experimental import pallas as pl


def kernel(x, edge_index, batch, Wm, bm, Wu, bu):
    raise NotImplementedError("write your pallas kernel here")


# speedup vs baseline: 3.9622x; 3.9622x over previous
import functools

import jax
import jax.numpy as jnp
from jax import lax
from jax.experimental import pallas as pl
from jax.experimental.pallas import tpu as pltpu
from jax.experimental.pallas import tpu_sc as plsc

N = 10000
E = 320000
D = 128
G = 64
NC = 2
NS = 16
NW = NC * NS
C = 128
NCH = 80
E_PAD = NW * NCH * C
NP = 10240
SLAB = NP // NS
BLK = 512
GRID = NP // BLK
CW = 128


@functools.lru_cache(maxsize=1)
def _sc_kernels():
    mesh = plsc.VectorSubcoreMesh(core_axis_name="c", subcore_axis_name="s",
                                  num_cores=NC, num_subcores=NS)

    @functools.partial(
        pl.kernel,
        out_type=jax.ShapeDtypeStruct((NC, NP, CW), jnp.float32),
        mesh=mesh,
        scratch_types=[
            pltpu.VMEM_SHARED((NP, CW), jnp.float32),
            pltpu.VMEM((NCH, C), jnp.int32),
            pltpu.VMEM((C, CW), jnp.float32),
        ],
    )
    def sc_indeg(dst_hbm, ones_hbm, zeros_hbm, out_hbm, table, dst_v, ones_v):
        c = lax.axis_index("c")
        s = lax.axis_index("s")
        wid = c * NS + s
        pltpu.sync_copy(zeros_hbm, table.at[pl.ds(s * SLAB, SLAB)])
        pltpu.sync_copy(ones_hbm, ones_v)
        pltpu.sync_copy(dst_hbm.at[wid], dst_v)
        plsc.subcore_barrier()

        def body(j, carry):
            pltpu.sync_copy(ones_v, table.at[dst_v.at[j]], add=True)
            return carry

        lax.fori_loop(0, NCH, body, 0)
        plsc.subcore_barrier()
        pltpu.sync_copy(table.at[pl.ds(s * SLAB, SLAB)],
                        out_hbm.at[c, pl.ds(s * SLAB, SLAB)])

    @functools.partial(
        pl.kernel,
        out_type=jax.ShapeDtypeStruct((NC, NP, D), jnp.float32),
        mesh=mesh,
        scratch_types=[
            pltpu.VMEM_SHARED((NP, D), jnp.float32),
            pltpu.VMEM((NCH, C), jnp.int32),
            pltpu.VMEM((NCH, C), jnp.int32),
            pltpu.VMEM((C, D), jnp.float32),
            pltpu.SemaphoreType.DMA,
        ],
    )
    def sc_agg(msg_hbm, src_hbm, dst_hbm, zeros_hbm, out_hbm,
               table, src_v, dst_v, rows, sem):
        c = lax.axis_index("c")
        s = lax.axis_index("s")
        wid = c * NS + s
        pltpu.sync_copy(zeros_hbm, table.at[pl.ds(s * SLAB, SLAB)])
        pltpu.sync_copy(src_hbm.at[wid], src_v)
        pltpu.sync_copy(dst_hbm.at[wid], dst_v)
        plsc.subcore_barrier()

        def body(j, carry):
            pltpu.async_copy(msg_hbm.at[src_v.at[j]], rows, sem).wait()
            pltpu.sync_copy(rows, table.at[dst_v.at[j]], add=True)
            return carry

        lax.fori_loop(0, NCH, body, 0)
        plsc.subcore_barrier()
        pltpu.sync_copy(table.at[pl.ds(s * SLAB, SLAB)],
                        out_hbm.at[c, pl.ds(s * SLAB, SLAB)])

    return sc_indeg, sc_agg


def _dotT(a, w):
    return lax.dot_general(a, w, (((1,), (1,)), ((), ())),
                           preferred_element_type=jnp.float32)


def _b1_body(cnt_ref, wu0_ref, bu0_ref, bm0_ref, wm1_ref, bm1_ref,
             s1_ref, m1_ref):
    m0 = jnp.maximum(bm0_ref[...], 0.0)
    v0 = _dotT(m0, wu0_ref[...])
    cnt = cnt_ref[0, :, 0:1] + cnt_ref[1, :, 0:1]
    s1 = jnp.maximum(cnt * v0 + bu0_ref[...], 0.0)
    s1_ref[...] = s1
    m1_ref[...] = jnp.maximum(_dotT(s1, wm1_ref[...]) + bm1_ref[...], 0.0)


def _bmid_body(sp_ref, agg_ref, wu_ref, bu_ref, wm_ref, bm_ref,
               s_ref, m_ref):
    agg = agg_ref[0] + agg_ref[1]
    s = sp_ref[...] + jnp.maximum(_dotT(agg, wu_ref[...]) + bu_ref[...], 0.0)
    s_ref[...] = s
    m_ref[...] = jnp.maximum(_dotT(s, wm_ref[...]) + bm_ref[...], 0.0)


def _b4_body(sp_ref, agg_ref, wu_ref, bu_ref, batch_ref, graph_ref):
    agg = agg_ref[0] + agg_ref[1]
    s4 = sp_ref[...] + jnp.maximum(_dotT(agg, wu_ref[...]) + bu_ref[...], 0.0)
    b = batch_ref[0, 0, :].reshape(1, BLK)
    gi = lax.broadcasted_iota(jnp.int32, (G, BLK), 0)
    onehot = (b == gi).astype(jnp.float32)
    contrib = lax.dot_general(onehot, s4, (((1,), (0,)), ((), ())),
                              preferred_element_type=jnp.float32)

    @pl.when(pl.program_id(0) == 0)
    def _():
        graph_ref[...] = jnp.zeros_like(graph_ref)

    graph_ref[...] += contrib


_W_SPEC = pl.BlockSpec((D, D), lambda i: (0, 0))
_B_SPEC = pl.BlockSpec((1, D), lambda i: (0, 0))
_ROW_SPEC = pl.BlockSpec((BLK, D), lambda i: (i, 0))
_AGG_SPEC = pl.BlockSpec((NC, BLK, D), lambda i: (0, i, 0))

_b1_call = pl.pallas_call(
    _b1_body,
    grid=(GRID,),
    in_specs=[pl.BlockSpec((NC, BLK, CW), lambda i: (0, i, 0)),
              _W_SPEC, _B_SPEC, _B_SPEC, _W_SPEC, _B_SPEC],
    out_specs=[_ROW_SPEC, _ROW_SPEC],
    out_shape=[jax.ShapeDtypeStruct((NP, D), jnp.float32),
               jax.ShapeDtypeStruct((NP, D), jnp.float32)],
)

_bmid_call = pl.pallas_call(
    _bmid_body,
    grid=(GRID,),
    in_specs=[_ROW_SPEC, _AGG_SPEC, _W_SPEC, _B_SPEC, _W_SPEC, _B_SPEC],
    out_specs=[_ROW_SPEC, _ROW_SPEC],
    out_shape=[jax.ShapeDtypeStruct((NP, D), jnp.float32),
               jax.ShapeDtypeStruct((NP, D), jnp.float32)],
)

_b4_call = pl.pallas_call(
    _b4_body,
    grid=(GRID,),
    in_specs=[_ROW_SPEC, _AGG_SPEC, _W_SPEC, _B_SPEC,
              pl.BlockSpec((1, 1, BLK), lambda i: (i, 0, 0))],
    out_specs=pl.BlockSpec((G, D), lambda i: (0, 0)),
    out_shape=jax.ShapeDtypeStruct((G, D), jnp.float32),
)


def kernel(x, edge_index, batch, Wm, bm, Wu, bu):
    src = edge_index[0]
    dst = edge_index[1]
    pad = E_PAD - E
    src_r = jnp.concatenate([src, jnp.zeros((pad,), jnp.int32)]).reshape(NW, NCH, C)
    dst_r = jnp.concatenate([dst, jnp.full((pad,), N, jnp.int32)]).reshape(NW, NCH, C)
    batch_r = jnp.concatenate([batch, jnp.full((NP - N,), G, jnp.int32)]).reshape(GRID, 1, BLK)
    zeros_d = jnp.zeros((SLAB, D), jnp.float32)
    ones_c = jnp.ones((C, CW), jnp.float32)
    bmr = bm.reshape(4, 1, D)
    bur = bu.reshape(4, 1, D)

    sc_indeg, sc_agg = _sc_kernels()
    cnt = sc_indeg(dst_r, ones_c, zeros_d)
    s1, m1 = _b1_call(cnt, Wu[0], bur[0], bmr[0], Wm[1], bmr[1])
    agg1 = sc_agg(m1, src_r, dst_r, zeros_d)
    s2, m2 = _bmid_call(s1, agg1, Wu[1], bur[1], Wm[2], bmr[2])
    agg2 = sc_agg(m2, src_r, dst_r, zeros_d)
    s3, m3 = _bmid_call(s2, agg2, Wu[2], bur[2], Wm[3], bmr[3])
    agg3 = sc_agg(m3, src_r, dst_r, zeros_d)
    graph = _b4_call(s3, agg3, Wu[3], bur[3], batch_r)
    return graph


# speedup vs baseline: 4.4549x; 1.1244x over previous
import functools

import jax
import jax.numpy as jnp
from jax import lax
from jax.experimental import pallas as pl
from jax.experimental.pallas import tpu as pltpu
from jax.experimental.pallas import tpu_sc as plsc

N = 10000
E = 320000
D = 128
G = 64
NC = 2
NS = 16
NW = NC * NS
C = 128
NCH = 80
IB = 16
E_PAD = NW * NCH * C
NP = 10240
SLAB = NP // NS
BLK = 512
GRID = NP // BLK
CW = 128


@functools.lru_cache(maxsize=1)
def _sc_kernels():
    mesh = plsc.VectorSubcoreMesh(core_axis_name="c", subcore_axis_name="s",
                                  num_cores=NC, num_subcores=NS)

    @functools.partial(
        pl.kernel,
        out_type=jax.ShapeDtypeStruct((NC, NP, CW), jnp.float32),
        mesh=mesh,
        scratch_types=[
            pltpu.VMEM_SHARED((NP, CW), jnp.float32),
            pltpu.VMEM((NCH, C), jnp.int32),
            pltpu.VMEM((C, CW), jnp.float32),
        ],
    )
    def sc_indeg(dst_hbm, ones_hbm, zeros_hbm, out_hbm, table, dst_v, ones_v):
        c = lax.axis_index("c")
        s = lax.axis_index("s")
        wid = c * NS + s
        pltpu.sync_copy(zeros_hbm, table.at[pl.ds(s * SLAB, SLAB)])
        pltpu.sync_copy(ones_hbm, ones_v)
        pltpu.sync_copy(dst_hbm.at[wid], dst_v)
        plsc.subcore_barrier()

        def body(j, carry):
            pltpu.sync_copy(ones_v, table.at[dst_v.at[j]], add=True)
            return carry

        lax.fori_loop(0, NCH, body, 0)
        plsc.subcore_barrier()
        pltpu.sync_copy(table.at[pl.ds(s * SLAB, SLAB)],
                        out_hbm.at[c, pl.ds(s * SLAB, SLAB)])

    @functools.partial(
        pl.kernel,
        out_type=jax.ShapeDtypeStruct((NC, NP, D), jnp.float32),
        mesh=mesh,
        scratch_types=[
            pltpu.VMEM_SHARED((NP, D), jnp.float32),
            pltpu.VMEM((IB, C), jnp.int32),
            pltpu.VMEM((IB, C), jnp.int32),
            pltpu.VMEM((C, D), jnp.float32),
            pltpu.VMEM((C, D), jnp.float32),
            pltpu.SemaphoreType.DMA,
            pltpu.SemaphoreType.DMA,
        ],
    )
    def sc_agg(msg_hbm, src_hbm, dst_hbm, zeros_hbm, out_hbm,
               table, src_v, dst_v, rows0, rows1, sem0, sem1):
        c = lax.axis_index("c")
        s = lax.axis_index("s")
        wid = c * NS + s
        pltpu.sync_copy(zeros_hbm, table.at[pl.ds(s * SLAB, SLAB)])
        plsc.subcore_barrier()

        def gather(j, buf, sem):
            return pltpu.make_async_copy(msg_hbm.at[src_v.at[j]], buf, sem)

        def group(g, carry):
            pltpu.sync_copy(src_hbm.at[wid, pl.ds(g * IB, IB)], src_v)
            pltpu.sync_copy(dst_hbm.at[wid, pl.ds(g * IB, IB)], dst_v)
            gather(0, rows0, sem0).start()

            def body(jj, inner):
                j0 = 2 * jj
                j1 = j0 + 1
                gather(j1, rows1, sem1).start()
                gather(j0, rows0, sem0).wait()
                pltpu.sync_copy(rows0, table.at[dst_v.at[j0]], add=True)

                @pl.when(jj < IB // 2 - 1)
                def _():
                    gather(j0 + 2, rows0, sem0).start()

                gather(j1, rows1, sem1).wait()
                pltpu.sync_copy(rows1, table.at[dst_v.at[j1]], add=True)
                return inner

            lax.fori_loop(0, IB // 2, body, carry)
            return carry

        lax.fori_loop(0, NCH // IB, group, 0)
        plsc.subcore_barrier()
        pltpu.sync_copy(table.at[pl.ds(s * SLAB, SLAB)],
                        out_hbm.at[c, pl.ds(s * SLAB, SLAB)])

    return sc_indeg, sc_agg


def _dotT(a, w):
    return lax.dot_general(a, w, (((1,), (1,)), ((), ())),
                           preferred_element_type=jnp.float32)


def _b1_body(cnt_ref, wu0_ref, bu0_ref, bm0_ref, wm1_ref, bm1_ref,
             s1_ref, m1_ref):
    m0 = jnp.maximum(bm0_ref[...], 0.0)
    v0 = _dotT(m0, wu0_ref[...])
    cnt = cnt_ref[0, :, 0:1] + cnt_ref[1, :, 0:1]
    s1 = jnp.maximum(cnt * v0 + bu0_ref[...], 0.0)
    s1_ref[...] = s1
    m1_ref[...] = jnp.maximum(_dotT(s1, wm1_ref[...]) + bm1_ref[...], 0.0)


def _bmid_body(sp_ref, agg_ref, wu_ref, bu_ref, wm_ref, bm_ref,
               s_ref, m_ref):
    agg = agg_ref[0] + agg_ref[1]
    s = sp_ref[...] + jnp.maximum(_dotT(agg, wu_ref[...]) + bu_ref[...], 0.0)
    s_ref[...] = s
    m_ref[...] = jnp.maximum(_dotT(s, wm_ref[...]) + bm_ref[...], 0.0)


def _b4_body(sp_ref, agg_ref, wu_ref, bu_ref, batch_ref, graph_ref):
    agg = agg_ref[0] + agg_ref[1]
    s4 = sp_ref[...] + jnp.maximum(_dotT(agg, wu_ref[...]) + bu_ref[...], 0.0)
    b = batch_ref[0, 0, :].reshape(1, BLK)
    gi = lax.broadcasted_iota(jnp.int32, (G, BLK), 0)
    onehot = (b == gi).astype(jnp.float32)
    contrib = lax.dot_general(onehot, s4, (((1,), (0,)), ((), ())),
                              preferred_element_type=jnp.float32)

    @pl.when(pl.program_id(0) == 0)
    def _():
        graph_ref[...] = jnp.zeros_like(graph_ref)

    graph_ref[...] += contrib


_W_SPEC = pl.BlockSpec((D, D), lambda i: (0, 0))
_B_SPEC = pl.BlockSpec((1, D), lambda i: (0, 0))
_ROW_SPEC = pl.BlockSpec((BLK, D), lambda i: (i, 0))
_AGG_SPEC = pl.BlockSpec((NC, BLK, D), lambda i: (0, i, 0))

_b1_call = pl.pallas_call(
    _b1_body,
    grid=(GRID,),
    in_specs=[pl.BlockSpec((NC, BLK, CW), lambda i: (0, i, 0)),
              _W_SPEC, _B_SPEC, _B_SPEC, _W_SPEC, _B_SPEC],
    out_specs=[_ROW_SPEC, _ROW_SPEC],
    out_shape=[jax.ShapeDtypeStruct((NP, D), jnp.float32),
               jax.ShapeDtypeStruct((NP, D), jnp.float32)],
)

_bmid_call = pl.pallas_call(
    _bmid_body,
    grid=(GRID,),
    in_specs=[_ROW_SPEC, _AGG_SPEC, _W_SPEC, _B_SPEC, _W_SPEC, _B_SPEC],
    out_specs=[_ROW_SPEC, _ROW_SPEC],
    out_shape=[jax.ShapeDtypeStruct((NP, D), jnp.float32),
               jax.ShapeDtypeStruct((NP, D), jnp.float32)],
)

_b4_call = pl.pallas_call(
    _b4_body,
    grid=(GRID,),
    in_specs=[_ROW_SPEC, _AGG_SPEC, _W_SPEC, _B_SPEC,
              pl.BlockSpec((1, 1, BLK), lambda i: (i, 0, 0))],
    out_specs=pl.BlockSpec((G, D), lambda i: (0, 0)),
    out_shape=jax.ShapeDtypeStruct((G, D), jnp.float32),
)


def kernel(x, edge_index, batch, Wm, bm, Wu, bu):
    src = edge_index[0]
    dst = edge_index[1]
    pad = E_PAD - E
    src_r = jnp.concatenate([src, jnp.zeros((pad,), jnp.int32)]).reshape(NW, NCH, C)
    dst_r = jnp.concatenate([dst, jnp.full((pad,), N, jnp.int32)]).reshape(NW, NCH, C)
    batch_r = jnp.concatenate([batch, jnp.full((NP - N,), G, jnp.int32)]).reshape(GRID, 1, BLK)
    zeros_d = jnp.zeros((SLAB, D), jnp.float32)
    ones_c = jnp.ones((C, CW), jnp.float32)
    bmr = bm.reshape(4, 1, D)
    bur = bu.reshape(4, 1, D)

    sc_indeg, sc_agg = _sc_kernels()
    cnt = sc_indeg(dst_r, ones_c, zeros_d)
    s1, m1 = _b1_call(cnt, Wu[0], bur[0], bmr[0], Wm[1], bmr[1])
    agg1 = sc_agg(m1, src_r, dst_r, zeros_d)
    s2, m2 = _bmid_call(s1, agg1, Wu[1], bur[1], Wm[2], bmr[2])
    agg2 = sc_agg(m2, src_r, dst_r, zeros_d)
    s3, m3 = _bmid_call(s2, agg2, Wu[2], bur[2], Wm[3], bmr[3])
    agg3 = sc_agg(m3, src_r, dst_r, zeros_d)
    graph = _b4_call(s3, agg3, Wu[3], bur[3], batch_r)
    return graph


# speedup vs baseline: 4.6619x; 1.0465x over previous
import functools

import jax
import jax.numpy as jnp
from jax import lax
from jax.experimental import pallas as pl
from jax.experimental.pallas import tpu as pltpu
from jax.experimental.pallas import tpu_sc as plsc

N = 10000
E = 320000
D = 128
G = 64
NC = 2
NS = 16
NW = NC * NS
C = 128
NCH = 80
NCHA = 120
NCHB = 40
IB = 40
TOTCH = NS * (NCHA + NCHB)
E_PAD = TOTCH * C
NP = 10240
SLAB = NP // NS
BLK = 512
GRID = NP // BLK
CW = 128


@functools.lru_cache(maxsize=1)
def _sc_kernels():
    mesh = plsc.VectorSubcoreMesh(core_axis_name="c", subcore_axis_name="s",
                                  num_cores=NC, num_subcores=NS)

    @functools.partial(
        pl.kernel,
        out_type=jax.ShapeDtypeStruct((NC, NP, CW), jnp.float32),
        mesh=mesh,
        scratch_types=[
            pltpu.VMEM_SHARED((NP, CW), jnp.float32),
            pltpu.VMEM((NCH, C), jnp.int32),
            pltpu.VMEM((C, CW), jnp.float32),
        ],
    )
    def sc_indeg(dst_hbm, ones_hbm, zeros_hbm, out_hbm, table, dst_v, ones_v):
        c = lax.axis_index("c")
        s = lax.axis_index("s")
        wid = c * NS + s
        pltpu.sync_copy(zeros_hbm, table.at[pl.ds(s * SLAB, SLAB)])
        pltpu.sync_copy(ones_hbm, ones_v)
        pltpu.sync_copy(dst_hbm.at[pl.ds(wid * NCH, NCH)], dst_v)
        plsc.subcore_barrier()

        def body(j, carry):
            pltpu.sync_copy(ones_v, table.at[dst_v.at[j]], add=True)
            return carry

        lax.fori_loop(0, NCH, body, 0)
        plsc.subcore_barrier()
        pltpu.sync_copy(table.at[pl.ds(s * SLAB, SLAB)],
                        out_hbm.at[c, pl.ds(s * SLAB, SLAB)])

    @functools.partial(
        pl.kernel,
        out_type=jax.ShapeDtypeStruct((NC, NP, D), jnp.float32),
        mesh=mesh,
        scratch_types=[
            pltpu.VMEM_SHARED((NP, D), jnp.float32),
            pltpu.VMEM((IB, C), jnp.int32),
            pltpu.VMEM((IB, C), jnp.int32),
            pltpu.VMEM((C, D), jnp.float32),
            pltpu.VMEM((C, D), jnp.float32),
            pltpu.SemaphoreType.DMA,
            pltpu.SemaphoreType.DMA,
        ],
    )
    def sc_agg(msg_hbm, src_hbm, dst_hbm, zeros_hbm, out_hbm,
               table, src_v, dst_v, rows0, rows1, sem0, sem1):
        c = lax.axis_index("c")
        s = lax.axis_index("s")
        base = jnp.where(c == 0, s * NCHA, NS * NCHA + s * NCHB)
        ngroups = jnp.where(c == 0, NCHA // IB, NCHB // IB)
        pltpu.sync_copy(zeros_hbm, table.at[pl.ds(s * SLAB, SLAB)])
        plsc.subcore_barrier()

        def gather(j, buf, sem):
            return pltpu.make_async_copy(msg_hbm.at[src_v.at[j]], buf, sem)

        def group(g, carry):
            pltpu.sync_copy(src_hbm.at[pl.ds(base + g * IB, IB)], src_v)
            pltpu.sync_copy(dst_hbm.at[pl.ds(base + g * IB, IB)], dst_v)
            gather(0, rows0, sem0).start()

            def body(jj, inner):
                j0 = 2 * jj
                j1 = j0 + 1
                gather(j1, rows1, sem1).start()
                gather(j0, rows0, sem0).wait()
                pltpu.sync_copy(rows0, table.at[dst_v.at[j0]], add=True)

                @pl.when(jj < IB // 2 - 1)
                def _():
                    gather(j0 + 2, rows0, sem0).start()

                gather(j1, rows1, sem1).wait()
                pltpu.sync_copy(rows1, table.at[dst_v.at[j1]], add=True)
                return inner

            lax.fori_loop(0, IB // 2, body, carry)
            return carry

        lax.fori_loop(0, ngroups, group, 0)
        plsc.subcore_barrier()
        pltpu.sync_copy(table.at[pl.ds(s * SLAB, SLAB)],
                        out_hbm.at[c, pl.ds(s * SLAB, SLAB)])

    return sc_indeg, sc_agg


def _dotT(a, w):
    return lax.dot_general(a, w, (((1,), (1,)), ((), ())),
                           preferred_element_type=jnp.float32)


def _b1_body(cnt_ref, wu0_ref, bu0_ref, bm0_ref, wm1_ref, bm1_ref,
             s1_ref, m1_ref):
    m0 = jnp.maximum(bm0_ref[...], 0.0)
    v0 = _dotT(m0, wu0_ref[...])
    cnt = cnt_ref[0, :, 0:1] + cnt_ref[1, :, 0:1]
    s1 = jnp.maximum(cnt * v0 + bu0_ref[...], 0.0)
    s1_ref[...] = s1
    m1_ref[...] = jnp.maximum(_dotT(s1, wm1_ref[...]) + bm1_ref[...], 0.0)


def _bmid_body(sp_ref, agg_ref, wu_ref, bu_ref, wm_ref, bm_ref,
               s_ref, m_ref):
    agg = agg_ref[0] + agg_ref[1]
    s = sp_ref[...] + jnp.maximum(_dotT(agg, wu_ref[...]) + bu_ref[...], 0.0)
    s_ref[...] = s
    m_ref[...] = jnp.maximum(_dotT(s, wm_ref[...]) + bm_ref[...], 0.0)


def _b4_body(sp_ref, agg_ref, wu_ref, bu_ref, batch_ref, graph_ref):
    agg = agg_ref[0] + agg_ref[1]
    s4 = sp_ref[...] + jnp.maximum(_dotT(agg, wu_ref[...]) + bu_ref[...], 0.0)
    b = batch_ref[0, 0, :].reshape(1, BLK)
    gi = lax.broadcasted_iota(jnp.int32, (G, BLK), 0)
    onehot = (b == gi).astype(jnp.float32)
    contrib = lax.dot_general(onehot, s4, (((1,), (0,)), ((), ())),
                              preferred_element_type=jnp.float32)

    @pl.when(pl.program_id(0) == 0)
    def _():
        graph_ref[...] = jnp.zeros_like(graph_ref)

    graph_ref[...] += contrib


_W_SPEC = pl.BlockSpec((D, D), lambda i: (0, 0))
_B_SPEC = pl.BlockSpec((1, D), lambda i: (0, 0))
_ROW_SPEC = pl.BlockSpec((BLK, D), lambda i: (i, 0))
_AGG_SPEC = pl.BlockSpec((NC, BLK, D), lambda i: (0, i, 0))

_b1_call = pl.pallas_call(
    _b1_body,
    grid=(GRID,),
    in_specs=[pl.BlockSpec((NC, BLK, CW), lambda i: (0, i, 0)),
              _W_SPEC, _B_SPEC, _B_SPEC, _W_SPEC, _B_SPEC],
    out_specs=[_ROW_SPEC, _ROW_SPEC],
    out_shape=[jax.ShapeDtypeStruct((NP, D), jnp.float32),
               jax.ShapeDtypeStruct((NP, D), jnp.float32)],
)

_bmid_call = pl.pallas_call(
    _bmid_body,
    grid=(GRID,),
    in_specs=[_ROW_SPEC, _AGG_SPEC, _W_SPEC, _B_SPEC, _W_SPEC, _B_SPEC],
    out_specs=[_ROW_SPEC, _ROW_SPEC],
    out_shape=[jax.ShapeDtypeStruct((NP, D), jnp.float32),
               jax.ShapeDtypeStruct((NP, D), jnp.float32)],
)

_b4_call = pl.pallas_call(
    _b4_body,
    grid=(GRID,),
    in_specs=[_ROW_SPEC, _AGG_SPEC, _W_SPEC, _B_SPEC,
              pl.BlockSpec((1, 1, BLK), lambda i: (i, 0, 0))],
    out_specs=pl.BlockSpec((G, D), lambda i: (0, 0)),
    out_shape=jax.ShapeDtypeStruct((G, D), jnp.float32),
)


def kernel(x, edge_index, batch, Wm, bm, Wu, bu):
    src = edge_index[0]
    dst = edge_index[1]
    pad = E_PAD - E
    src_r = jnp.concatenate([src, jnp.zeros((pad,), jnp.int32)]).reshape(TOTCH, C)
    dst_r = jnp.concatenate([dst, jnp.full((pad,), N, jnp.int32)]).reshape(TOTCH, C)
    batch_r = jnp.concatenate([batch, jnp.full((NP - N,), G, jnp.int32)]).reshape(GRID, 1, BLK)
    zeros_d = jnp.zeros((SLAB, D), jnp.float32)
    ones_c = jnp.ones((C, CW), jnp.float32)
    bmr = bm.reshape(4, 1, D)
    bur = bu.reshape(4, 1, D)

    sc_indeg, sc_agg = _sc_kernels()
    cnt = sc_indeg(dst_r, ones_c, zeros_d)
    s1, m1 = _b1_call(cnt, Wu[0], bur[0], bmr[0], Wm[1], bmr[1])
    agg1 = sc_agg(m1, src_r, dst_r, zeros_d)
    s2, m2 = _bmid_call(s1, agg1, Wu[1], bur[1], Wm[2], bmr[2])
    agg2 = sc_agg(m2, src_r, dst_r, zeros_d)
    s3, m3 = _bmid_call(s2, agg2, Wu[2], bur[2], Wm[3], bmr[3])
    agg3 = sc_agg(m3, src_r, dst_r, zeros_d)
    graph = _b4_call(s3, agg3, Wu[3], bur[3], batch_r)
    return graph
